# baseline (speedup 1.0000x reference)
import functools

import jax
import jax.numpy as jnp
from jax import lax
from jax.experimental import pallas as pl
from jax.experimental.pallas import tpu as pltpu
from jax.experimental.pallas import tpu_sc as plsc

MAXL = 8
B, S, D = 8, 2048, 128
NW = 32
Q = 4
QS = S // Q
TR = 128
NT = QS // TR
NIN = QS + MAXL
NG = D // 16

_BASE = []
_off = 0
for _L in range(1, MAXL + 1):
    _BASE.append(_off)
    _off += S - _L
OUT_S = _off

_mesh = plsc.VectorSubcoreMesh(core_axis_name="c", subcore_axis_name="s")


@functools.partial(
    pl.kernel,
    out_type=jax.ShapeDtypeStruct((B, OUT_S, D), jnp.float32),
    mesh=_mesh,
    compiler_params=pltpu.CompilerParams(use_tc_tiling_on_sc=False),
    scratch_types=[
        pltpu.VMEM((NIN, D), jnp.float32),
        pltpu.VMEM((2, TR, D), jnp.float32),
        pltpu.SemaphoreType.DMA,
        pltpu.SemaphoreType.DMA,
        pltpu.SemaphoreType.DMA,
    ],
)
def _span_kernel(t_hbm, out_hbm, pbuf, stage, sem_in, sem0, sem1):
    wid = lax.axis_index("s") * 2 + lax.axis_index("c")
    b = wid // Q
    q = wid % Q
    s0 = q * QS
    ls = pl.multiple_of(jnp.where(q == Q - 1, S - NIN, s0), 8)
    zeros = jnp.zeros((16,), jnp.float32)
    sems = (sem0, sem1)

    pltpu.async_copy(t_hbm.at[b, pl.ds(ls, NIN)], pbuf, sem_in).wait()

    for g in range(NG):
        pbuf[0, pl.ds(g * 16, 16)] = zeros

    def pfx(j, c):
        for g in range(NG):
            sl = pl.ds(g * 16, 16)
            pbuf[j, sl] = pbuf[j, sl] + pbuf[j - 1, sl]
        return c

    lax.fori_loop(1, NIN, pfx, 0)

    pending = [None, None]
    k = 0
    for L in range(1, MAXL + 1):
        for t in (NT - 1,) + tuple(range(NT - 1)):
            if t == NT - 1:
                i0 = jnp.where(q == Q - 1, S - L - TR, s0 + t * TR)
            else:
                i0 = s0 + t * TR
            m0 = i0 - ls
            p = k % 2
            k += 1
            if pending[p] is not None:
                pending[p].wait()

            def gbody(g, c, L=L, p=p, m0=m0):
                sl = pl.ds(g * 16, 16)
                ring = tuple(pbuf[m0 + j, sl] for j in range(L))

                def rbody(r, ring):
                    new = pbuf[m0 + r + L, sl]
                    stage[p, r, sl] = new - ring[0]
                    return ring[1:] + (new,)

                lax.fori_loop(0, TR, rbody, ring, unroll=8)
                return c

            lax.fori_loop(0, NG, gbody, 0)
            row0 = _BASE[L - 1] + i0
            pending[p] = pltpu.async_copy(
                stage.at[p], out_hbm.at[b, pl.ds(row0, TR)], sems[p])

    for p in (0, 1):
        if pending[p] is not None:
            pending[p].wait()


def kernel(tensor):
    return _span_kernel(tensor)


# speedup vs baseline: 1.4068x; 1.4068x over previous
import functools

import jax
import jax.numpy as jnp
from jax import lax
from jax.experimental import pallas as pl
from jax.experimental.pallas import tpu as pltpu
from jax.experimental.pallas import tpu_sc as plsc

MAXL = 8
B, S, D = 8, 2048, 128
NW = 32
W = S // NW
NIN = W + MAXL
TR = 16
NT = W // TR
NG = D // 16

_BASE = []
_off = 0
for _L in range(1, MAXL + 1):
    _BASE.append(_off)
    _off += S - _L
OUT_S = _off

_mesh = plsc.VectorSubcoreMesh(core_axis_name="c", subcore_axis_name="s")


@functools.partial(
    pl.kernel,
    out_type=jax.ShapeDtypeStruct((OUT_S, B, D), jnp.float32),
    mesh=_mesh,
    scratch_types=[
        pltpu.VMEM((B, NIN, D), jnp.float32),
        pltpu.VMEM((2, TR, B, D), jnp.float32),
        pltpu.SemaphoreType.DMA,
        pltpu.SemaphoreType.DMA,
        pltpu.SemaphoreType.DMA,
    ],
)
def _span_kernel(t_hbm, out_hbm, pbuf, stage, sem_in, sem0, sem1):
    wid = lax.axis_index("s") * 2 + lax.axis_index("c")
    s0 = wid * W
    shifted = wid == NW - 1
    ls = pl.multiple_of(jnp.where(shifted, S - NIN, s0), 8)
    zeros = jnp.zeros((16,), jnp.float32)
    sems = (sem0, sem1)

    in_handles = []
    for b in range(B):
        in_handles.append(pltpu.async_copy(
            t_hbm.at[b, pl.ds(ls, NIN)], pbuf.at[b], sem_in))
    for h in in_handles:
        h.wait()

    for b in range(B):
        for g in range(NG):
            pbuf[b, 0, pl.ds(g * 16, 16)] = zeros

    def pfx(j, c):
        for b in range(B):
            for g in range(NG):
                sl = pl.ds(g * 16, 16)
                pbuf[b, j, sl] = pbuf[b, j, sl] + pbuf[b, j - 1, sl]
        return c

    lax.fori_loop(1, NIN, pfx, 0)

    pending = [None, None]
    k = 0
    for L in range(1, MAXL + 1):
        w0 = jnp.where(shifted, S - L - W, s0)
        for t in range(NT):
            i0 = w0 + t * TR
            m0 = i0 - ls
            p = k % 2
            k += 1
            if pending[p] is not None:
                pending[p].wait()

            def bbody(b, c, L=L, p=p, m0=m0):
                def gbody(g, c2, b=b):
                    sl = pl.ds(g * 16, 16)
                    ring = tuple(pbuf[b, m0 + j, sl] for j in range(L))

                    def rbody(r, ring):
                        new = pbuf[b, m0 + r + L, sl]
                        stage[p, r, b, sl] = new - ring[0]
                        return ring[1:] + (new,)

                    lax.fori_loop(0, TR, rbody, ring, unroll=8)
                    return c2

                lax.fori_loop(0, NG, gbody, 0)
                return c

            lax.fori_loop(0, B, bbody, 0)
            pending[p] = pltpu.async_copy(
                stage.at[p], out_hbm.at[pl.ds(_BASE[L - 1] + i0, TR)],
                sems[p])

    for p in (0, 1):
        if pending[p] is not None:
            pending[p].wait()


def kernel(tensor):
    return jnp.transpose(_span_kernel(tensor), (1, 0, 2))


# speedup vs baseline: 1.4852x; 1.0558x over previous
import functools

import jax
import jax.numpy as jnp
from jax import lax
from jax.experimental import pallas as pl
from jax.experimental.pallas import tpu as pltpu
from jax.experimental.pallas import tpu_sc as plsc

MAXL = 8
B, S, D = 8, 2048, 128
NW = 32
W = S // NW
HW = W // 2
NIN = HW + MAXL
NG = D // 16

_BASE = []
_off = 0
for _L in range(1, MAXL + 1):
    _BASE.append(_off)
    _off += S - _L
OUT_S = _off

_mesh = plsc.VectorSubcoreMesh(core_axis_name="c", subcore_axis_name="s")


@functools.partial(
    pl.kernel,
    out_type=jax.ShapeDtypeStruct((OUT_S, B, D), jnp.float32),
    mesh=_mesh,
    scratch_types=[
        pltpu.VMEM((B, NIN, D), jnp.float32),
        pltpu.VMEM((2, HW, B, D), jnp.float32),
        pltpu.SemaphoreType.DMA,
        pltpu.SemaphoreType.DMA,
        pltpu.SemaphoreType.DMA,
    ],
)
def _span_kernel(t_hbm, out_hbm, pbuf, stage, sem_in, sem0, sem1):
    wid = lax.axis_index("s") * 2 + lax.axis_index("c")
    zeros = jnp.zeros((16,), jnp.float32)
    sems = (sem0, sem1)
    pending = [None, None]
    k = 0
    for hh in (0, 1):
        s0 = wid * W + hh * HW
        if hh == 1:
            shifted = wid == NW - 1
            load_s0 = pl.multiple_of(jnp.where(shifted, S - NIN, s0), 8)
        else:
            shifted = None
            load_s0 = pl.multiple_of(s0, 8)

        in_handles = []
        for b in range(B):
            in_handles.append(pltpu.async_copy(
                t_hbm.at[b, pl.ds(load_s0, NIN)], pbuf.at[b], sem_in))
        for h in in_handles:
            h.wait()

        for b in range(B):
            for g in range(NG):
                pbuf[b, 0, pl.ds(g * 16, 16)] = zeros

        def pfx(j, c):
            for b in range(B):
                for g in range(NG):
                    sl = pl.ds(g * 16, 16)
                    pbuf[b, j, sl] = pbuf[b, j, sl] + pbuf[b, j - 1, sl]
            return c

        lax.fori_loop(1, NIN, pfx, 0)

        for L in range(1, MAXL + 1):
            if hh == 1:
                delta = jnp.where(shifted, MAXL - L, 0)
                row0 = _BASE[L - 1] + jnp.where(shifted, S - L - HW, s0)
            else:
                delta = 0
                row0 = _BASE[L - 1] + s0
            p = k % 2
            k += 1
            if pending[p] is not None:
                pending[p].wait()

            def bbody(b, c, L=L, p=p, delta=delta):
                def gbody(g, c2, b=b):
                    sl = pl.ds(g * 16, 16)
                    ring = tuple(pbuf[b, delta + j, sl] for j in range(L))

                    def rbody(r, ring):
                        new = pbuf[b, delta + r + L, sl]
                        stage[p, r, b, sl] = new - ring[0]
                        return ring[1:] + (new,)

                    lax.fori_loop(0, HW, rbody, ring, unroll=8)
                    return c2

                lax.fori_loop(0, NG, gbody, 0)
                return c

            lax.fori_loop(0, B, bbody, 0)
            pending[p] = pltpu.async_copy(
                stage.at[p], out_hbm.at[pl.ds(row0, HW)], sems[p])

    for p in (0, 1):
        if pending[p] is not None:
            pending[p].wait()


def kernel(tensor):
    return jnp.transpose(_span_kernel(tensor), (1, 0, 2))
